# baseline (speedup 1.0000x reference)
import functools

import jax
import jax.numpy as jnp
from jax import lax
from jax.experimental import pallas as pl
from jax.experimental.pallas import tpu as pltpu
from jax.experimental.pallas import tpu_sc as plsc

N = 10000
E = 320000
F = 128
HEADS = 8
C1 = 16
NC, NS, L = 2, 16, 16
NW = NC * NS
EPW = E // NW
K = 40
NWIN = EPW // K
RPT = 640
ZR = 16


def _edge_kernel_body(with_h, src_hbm, dst_hbm, a_hbm, b_hbm,
                      out_hbm, acc_sh, idxs, idxd, A, B, M, zb,
                      semg, sems, semi):
    c = lax.axis_index("c")
    s = lax.axis_index("s")
    wid = s * NC + c

    def zv(i, _):
        zb[i // (F // L), pl.ds((i % (F // L)) * L, L)] = jnp.zeros(
            (L,), jnp.float32)
        return ()
    lax.fori_loop(0, ZR * (F // L), zv, ())
    for r in range(RPT // ZR):
        off = s * RPT + r * ZR
        @pl.when(off <= N - ZR)
        def _():
            pltpu.sync_copy(zb, acc_sh.at[pl.ds(off, ZR)])
    plsc.subcore_barrier()

    base = wid * EPW

    def issue_idx(w):
        t = w % 3
        off = base + w * K
        pltpu.async_copy(src_hbm.at[pl.ds(off, K)], idxs.at[t], semi.at[t])
        pltpu.async_copy(dst_hbm.at[pl.ds(off, K)], idxd.at[t], semi.at[t])

    def drain_idx(w):
        t = w % 3
        off = base + w * K
        pltpu.make_async_copy(
            src_hbm.at[pl.ds(off, K)], idxs.at[t], semi.at[t]).wait()
        pltpu.make_async_copy(
            dst_hbm.at[pl.ds(off, K)], idxd.at[t], semi.at[t]).wait()

    def issue_gathers(w):
        p, t = w % 2, w % 3
        pltpu.async_copy(a_hbm.at[idxs.at[t]], A.at[p], semg.at[p])
        pltpu.async_copy(b_hbm.at[idxd.at[t]], B.at[p], semg.at[p])

    def drain_gathers(w):
        p, t = w % 2, w % 3
        pltpu.make_async_copy(a_hbm.at[idxs.at[t]], A.at[p], semg.at[p]).wait()
        pltpu.make_async_copy(b_hbm.at[idxd.at[t]], B.at[p], semg.at[p]).wait()

    def drain_scatter(w):
        p, t = w % 2, w % 3
        pltpu.make_async_copy(
            M.at[p], acc_sh.at[idxd.at[t]], sems.at[p]).wait()

    issue_idx(0)
    issue_idx(1)
    drain_idx(0)
    issue_gathers(0)

    def window(w, _):
        p, t = w % 2, w % 3

        @pl.when(w + 2 < NWIN)
        def _():
            issue_idx(w + 2)

        @pl.when(w + 1 < NWIN)
        def _():
            drain_idx(w + 1)
            issue_gathers(w + 1)
        drain_gathers(w)

        @pl.when(w >= 2)
        def _():
            drain_scatter(w - 2)

        @plsc.parallel_loop(0, K, unroll=2)
        def edge(e):
            for j in range(F // L):
                q = pl.ds(j * L, L)
                z = A[p, e, q] + B[p, e, q]
                z = jnp.maximum(z, 0.2 * z)
                ex = jnp.exp(z)
                if with_h:
                    M[p, e, q] = ex * A[p, e, pl.ds(F + j * L, L)]
                else:
                    M[p, e, q] = ex
        pltpu.async_copy(M.at[p], acc_sh.at[idxd.at[t]], sems.at[p], add=True)
        return ()
    lax.fori_loop(0, NWIN, window, ())

    for w in (NWIN - 2, NWIN - 1):
        drain_scatter(w)
    plsc.subcore_barrier()

    @pl.when(s < NS - 1)
    def _():
        rows = pl.ds(s * RPT, RPT)
        pltpu.sync_copy(acc_sh.at[rows], out_hbm.at[c, rows])

    @pl.when(s == NS - 1)
    def _():
        rows = pl.ds((NS - 1) * RPT, N - (NS - 1) * RPT)
        pltpu.sync_copy(acc_sh.at[rows], out_hbm.at[c, rows])


def _make_edge_call(with_h):
    CA = 2 * F if with_h else F
    mesh = plsc.VectorSubcoreMesh(core_axis_name="c", subcore_axis_name="s")
    scratch = [
        pltpu.VMEM_SHARED((N, F), jnp.float32),
        pltpu.VMEM((3, K), jnp.int32),
        pltpu.VMEM((3, K), jnp.int32),
        pltpu.VMEM((2, K, CA), jnp.float32),
        pltpu.VMEM((2, K, F), jnp.float32),
        pltpu.VMEM((2, K, F), jnp.float32),
        pltpu.VMEM((ZR, F), jnp.float32),
        pltpu.SemaphoreType.DMA((2,)),
        pltpu.SemaphoreType.DMA((2,)),
        pltpu.SemaphoreType.DMA((3,)),
    ]
    return pl.kernel(
        functools.partial(_edge_kernel_body, with_h),
        out_type=jax.ShapeDtypeStruct((NC, N, F), jnp.float32),
        mesh=mesh,
        scratch_types=scratch,
    )


_BM = 1000


def _proj_body(x_ref, w_ref, ah_ref, ax_ref, bx_ref):
    r = jnp.dot(x_ref[...], w_ref[...], preferred_element_type=jnp.float32, precision=lax.Precision.HIGHEST)
    ah_ref[:, :F] = r[:, F:2 * F]
    ah_ref[:, F:] = r[:, :F]
    ax_ref[...] = r[:, F:2 * F]
    bx_ref[...] = r[:, 2 * F:]


def _proj_call(x, wcat):
    g = x.shape[0] // _BM
    outs = [
        jax.ShapeDtypeStruct((x.shape[0], 2 * F), jnp.float32),
        jax.ShapeDtypeStruct((x.shape[0], F), jnp.float32),
        jax.ShapeDtypeStruct((x.shape[0], F), jnp.float32),
    ]
    return pl.pallas_call(
        _proj_body,
        grid=(g,),
        in_specs=[
            pl.BlockSpec((_BM, F), lambda i: (i, 0)),
            pl.BlockSpec((F, 3 * F), lambda i: (0, 0)),
        ],
        out_specs=[
            pl.BlockSpec((_BM, 2 * F), lambda i: (i, 0)),
            pl.BlockSpec((_BM, F), lambda i: (i, 0)),
            pl.BlockSpec((_BM, F), lambda i: (i, 0)),
        ],
        out_shape=outs,
    )(x, wcat)


def _combine_proj_body(acc_ref, den_ref, b_ref, w_ref, ah_ref, ax_ref, bx_ref):
    num = acc_ref[0] + acc_ref[1]
    dexp = den_ref[0] + den_ref[1]
    g = jnp.maximum(num / (dexp + 1e-16) + b_ref[...], 0.0)
    r = jnp.dot(g, w_ref[...], preferred_element_type=jnp.float32, precision=lax.Precision.HIGHEST)
    ah_ref[:, :F] = r[:, F:2 * F]
    ah_ref[:, F:] = r[:, :F]
    ax_ref[...] = r[:, F:2 * F]
    bx_ref[...] = r[:, 2 * F:]


def _combine_proj_call(acc, den, b1, wcat):
    outs = [
        jax.ShapeDtypeStruct((N, 2 * F), jnp.float32),
        jax.ShapeDtypeStruct((N, F), jnp.float32),
        jax.ShapeDtypeStruct((N, F), jnp.float32),
    ]
    return pl.pallas_call(
        _combine_proj_body,
        grid=(N // _BM,),
        in_specs=[
            pl.BlockSpec((NC, _BM, F), lambda i: (0, i, 0)),
            pl.BlockSpec((NC, _BM, F), lambda i: (0, i, 0)),
            pl.BlockSpec((1, F), lambda i: (0, 0)),
            pl.BlockSpec((F, 3 * F), lambda i: (0, 0)),
        ],
        out_specs=[
            pl.BlockSpec((_BM, 2 * F), lambda i: (i, 0)),
            pl.BlockSpec((_BM, F), lambda i: (i, 0)),
            pl.BlockSpec((_BM, F), lambda i: (i, 0)),
        ],
        out_shape=outs,
    )(acc, den, b1, wcat)


def _final_body(acc_ref, den_ref, b2_ref, wc1_ref, bc1_ref, wc2_ref, bc2_ref,
                emb_ref, lg_ref):
    num = acc_ref[0] + acc_ref[1]
    dexp = den_ref[0] + den_ref[1]
    emb = num / (dexp + 1e-16) + b2_ref[...]
    emb_ref[...] = emb
    hc = jnp.maximum(
        jnp.dot(emb, wc1_ref[...], preferred_element_type=jnp.float32, precision=lax.Precision.HIGHEST)
        + bc1_ref[...], 0.0)
    lg_ref[...] = (jnp.dot(hc, wc2_ref[...], preferred_element_type=jnp.float32, precision=lax.Precision.HIGHEST)
                   + bc2_ref[...])


def _final_call(acc, den, b2, wc1, bc1, wc2p, bc2p):
    outs = [
        jax.ShapeDtypeStruct((N, F), jnp.float32),
        jax.ShapeDtypeStruct((N, F), jnp.float32),
    ]
    return pl.pallas_call(
        _final_body,
        grid=(N // _BM,),
        in_specs=[
            pl.BlockSpec((NC, _BM, F), lambda i: (0, i, 0)),
            pl.BlockSpec((NC, _BM, F), lambda i: (0, i, 0)),
            pl.BlockSpec((1, F), lambda i: (0, 0)),
            pl.BlockSpec((F, F), lambda i: (0, 0)),
            pl.BlockSpec((1, F), lambda i: (0, 0)),
            pl.BlockSpec((F, F), lambda i: (0, 0)),
            pl.BlockSpec((1, F), lambda i: (0, 0)),
        ],
        out_specs=[pl.BlockSpec((_BM, F), lambda i: (i, 0))] * 2,
        out_shape=outs,
    )(acc, den, b2, wc1, bc1, wc2p, bc2p)


def _fold_weights(W, att_src, att_dst, heads, ch):
    a_s = att_src.reshape(heads, ch)
    a_d = att_dst.reshape(heads, ch)
    eyeh = jnp.eye(heads, dtype=jnp.float32)
    p_s = (a_s[:, :, None, None] * eyeh[:, None, :, None]
           * jnp.ones((1, 1, 1, ch))).reshape(heads * ch, heads * ch)
    p_d = (a_d[:, :, None, None] * eyeh[:, None, :, None]
           * jnp.ones((1, 1, 1, ch))).reshape(heads * ch, heads * ch)
    mm = lambda a, b: jnp.dot(a, b, precision=lax.Precision.HIGHEST)
    return jnp.concatenate([W, mm(W, p_s), mm(W, p_d)], axis=1)


_num_call = None
_den_call = None


def _get_calls():
    global _num_call, _den_call
    if _num_call is None:
        _num_call = _make_edge_call(True)
        _den_call = _make_edge_call(False)
    return _num_call, _den_call


def kernel(x, edge_index, W1, att_src1, att_dst1, b1, W2, att_src2, att_dst2,
           b2, Wc1, bc1, Wc2, bc2):
    num_call, den_call = _get_calls()
    src = edge_index[0].astype(jnp.int32)
    dst = edge_index[1].astype(jnp.int32)

    wcat1 = _fold_weights(W1, att_src1, att_dst1, HEADS, C1)
    wcat2 = _fold_weights(W2, att_src2, att_dst2, 1, F)

    ah1, asx1, adx1 = _proj_call(x, wcat1)
    acc1 = num_call(src, dst, ah1, adx1)
    den1 = den_call(src, dst, asx1, adx1)

    ah2, asx2, adx2 = _combine_proj_call(acc1, den1, b1.reshape(1, F), wcat2)
    acc2 = num_call(src, dst, ah2, adx2)
    den2 = den_call(src, dst, asx2, adx2)

    wc2p = jnp.zeros((F, F), jnp.float32).at[:, :2].set(Wc2)
    bc2p = jnp.zeros((1, F), jnp.float32).at[0, :2].set(bc2)
    emb, logitsp = _final_call(acc2, den2, b2.reshape(1, F),
                               Wc1, bc1.reshape(1, F), wc2p, bc2p)
    return emb, logitsp[:, :2]


# speedup vs baseline: 1.0329x; 1.0329x over previous
import functools

import jax
import jax.numpy as jnp
from jax import lax
from jax.experimental import pallas as pl
from jax.experimental.pallas import tpu as pltpu
from jax.experimental.pallas import tpu_sc as plsc

N = 10000
E = 320000
F = 128
HEADS = 8
C1 = 16
NC, NS, L = 2, 16, 16
NW = NC * NS
EPW = E // NW
K = 40
NWIN = EPW // K
RPT = 640
ZR = 16


def _edge_kernel_body(with_h, src_hbm, dst_hbm, a_hbm, b_hbm,
                      out_hbm, acc_sh, idxs, idxd, A, B, M, zb,
                      semg, sems, semi):
    c = lax.axis_index("c")
    s = lax.axis_index("s")
    wid = s * NC + c

    def zv(i, _):
        zb[i // (F // L), pl.ds((i % (F // L)) * L, L)] = jnp.zeros(
            (L,), jnp.float32)
        return ()
    lax.fori_loop(0, ZR * (F // L), zv, ())
    for r in range(RPT // ZR):
        off = s * RPT + r * ZR
        @pl.when(off <= N - ZR)
        def _():
            pltpu.sync_copy(zb, acc_sh.at[pl.ds(off, ZR)])
    plsc.subcore_barrier()

    base = wid * EPW

    def issue_idx(w):
        t = w % 3
        off = base + w * K
        pltpu.async_copy(src_hbm.at[pl.ds(off, K)], idxs.at[t], semi.at[t])
        pltpu.async_copy(dst_hbm.at[pl.ds(off, K)], idxd.at[t], semi.at[t])

    def drain_idx(w):
        t = w % 3
        off = base + w * K
        pltpu.make_async_copy(
            src_hbm.at[pl.ds(off, K)], idxs.at[t], semi.at[t]).wait()
        pltpu.make_async_copy(
            dst_hbm.at[pl.ds(off, K)], idxd.at[t], semi.at[t]).wait()

    def issue_gathers(w):
        p, t = w % 2, w % 3
        pltpu.async_copy(a_hbm.at[idxs.at[t]], A.at[p], semg.at[p])
        pltpu.async_copy(b_hbm.at[idxd.at[t]], B.at[p], semg.at[p])

    def drain_gathers(w):
        p, t = w % 2, w % 3
        pltpu.make_async_copy(a_hbm.at[idxs.at[t]], A.at[p], semg.at[p]).wait()
        pltpu.make_async_copy(b_hbm.at[idxd.at[t]], B.at[p], semg.at[p]).wait()

    def drain_scatter(w):
        p, t = w % 2, w % 3
        pltpu.make_async_copy(
            M.at[p], acc_sh.at[idxd.at[t]], sems.at[p]).wait()

    issue_idx(0)
    issue_idx(1)
    drain_idx(0)
    issue_gathers(0)

    def window(w, _):
        p, t = w % 2, w % 3

        @pl.when(w + 2 < NWIN)
        def _():
            issue_idx(w + 2)

        @pl.when(w + 1 < NWIN)
        def _():
            drain_idx(w + 1)
            issue_gathers(w + 1)
        drain_gathers(w)

        @pl.when(w >= 2)
        def _():
            drain_scatter(w - 2)

        @plsc.parallel_loop(0, K * (F // L), unroll=8)
        def edge(i):
            e = i // (F // L)
            j = i % (F // L)
            q = pl.ds(j * L, L)
            z = A[p, e, q] + B[p, e, q]
            z = jnp.maximum(z, 0.2 * z)
            ex = jnp.exp(z)
            if with_h:
                M[p, e, q] = ex * A[p, e, pl.ds(F + j * L, L)]
            else:
                M[p, e, q] = ex
        pltpu.async_copy(M.at[p], acc_sh.at[idxd.at[t]], sems.at[p], add=True)
        return ()
    lax.fori_loop(0, NWIN, window, ())

    for w in (NWIN - 2, NWIN - 1):
        drain_scatter(w)
    plsc.subcore_barrier()

    @pl.when(s < NS - 1)
    def _():
        rows = pl.ds(s * RPT, RPT)
        pltpu.sync_copy(acc_sh.at[rows], out_hbm.at[c, rows])

    @pl.when(s == NS - 1)
    def _():
        rows = pl.ds((NS - 1) * RPT, N - (NS - 1) * RPT)
        pltpu.sync_copy(acc_sh.at[rows], out_hbm.at[c, rows])


def _make_edge_call(with_h):
    CA = 2 * F if with_h else F
    mesh = plsc.VectorSubcoreMesh(core_axis_name="c", subcore_axis_name="s")
    scratch = [
        pltpu.VMEM_SHARED((N, F), jnp.float32),
        pltpu.VMEM((3, K), jnp.int32),
        pltpu.VMEM((3, K), jnp.int32),
        pltpu.VMEM((2, K, CA), jnp.float32),
        pltpu.VMEM((2, K, F), jnp.float32),
        pltpu.VMEM((2, K, F), jnp.float32),
        pltpu.VMEM((ZR, F), jnp.float32),
        pltpu.SemaphoreType.DMA((2,)),
        pltpu.SemaphoreType.DMA((2,)),
        pltpu.SemaphoreType.DMA((3,)),
    ]
    return pl.kernel(
        functools.partial(_edge_kernel_body, with_h),
        out_type=jax.ShapeDtypeStruct((NC, N, F), jnp.float32),
        mesh=mesh,
        scratch_types=scratch,
    )


_BM = 1000


def _proj_body(x_ref, w_ref, ah_ref, ax_ref, bx_ref):
    r = jnp.dot(x_ref[...], w_ref[...], preferred_element_type=jnp.float32, precision=lax.Precision.HIGHEST)
    ah_ref[:, :F] = r[:, F:2 * F]
    ah_ref[:, F:] = r[:, :F]
    ax_ref[...] = r[:, F:2 * F]
    bx_ref[...] = r[:, 2 * F:]


def _proj_call(x, wcat):
    g = x.shape[0] // _BM
    outs = [
        jax.ShapeDtypeStruct((x.shape[0], 2 * F), jnp.float32),
        jax.ShapeDtypeStruct((x.shape[0], F), jnp.float32),
        jax.ShapeDtypeStruct((x.shape[0], F), jnp.float32),
    ]
    return pl.pallas_call(
        _proj_body,
        grid=(g,),
        in_specs=[
            pl.BlockSpec((_BM, F), lambda i: (i, 0)),
            pl.BlockSpec((F, 3 * F), lambda i: (0, 0)),
        ],
        out_specs=[
            pl.BlockSpec((_BM, 2 * F), lambda i: (i, 0)),
            pl.BlockSpec((_BM, F), lambda i: (i, 0)),
            pl.BlockSpec((_BM, F), lambda i: (i, 0)),
        ],
        out_shape=outs,
    )(x, wcat)


def _combine_proj_body(acc_ref, den_ref, b_ref, w_ref, ah_ref, ax_ref, bx_ref):
    num = acc_ref[0] + acc_ref[1]
    dexp = den_ref[0] + den_ref[1]
    g = jnp.maximum(num / (dexp + 1e-16) + b_ref[...], 0.0)
    r = jnp.dot(g, w_ref[...], preferred_element_type=jnp.float32, precision=lax.Precision.HIGHEST)
    ah_ref[:, :F] = r[:, F:2 * F]
    ah_ref[:, F:] = r[:, :F]
    ax_ref[...] = r[:, F:2 * F]
    bx_ref[...] = r[:, 2 * F:]


def _combine_proj_call(acc, den, b1, wcat):
    outs = [
        jax.ShapeDtypeStruct((N, 2 * F), jnp.float32),
        jax.ShapeDtypeStruct((N, F), jnp.float32),
        jax.ShapeDtypeStruct((N, F), jnp.float32),
    ]
    return pl.pallas_call(
        _combine_proj_body,
        grid=(N // _BM,),
        in_specs=[
            pl.BlockSpec((NC, _BM, F), lambda i: (0, i, 0)),
            pl.BlockSpec((NC, _BM, F), lambda i: (0, i, 0)),
            pl.BlockSpec((1, F), lambda i: (0, 0)),
            pl.BlockSpec((F, 3 * F), lambda i: (0, 0)),
        ],
        out_specs=[
            pl.BlockSpec((_BM, 2 * F), lambda i: (i, 0)),
            pl.BlockSpec((_BM, F), lambda i: (i, 0)),
            pl.BlockSpec((_BM, F), lambda i: (i, 0)),
        ],
        out_shape=outs,
    )(acc, den, b1, wcat)


def _final_body(acc_ref, den_ref, b2_ref, wc1_ref, bc1_ref, wc2_ref, bc2_ref,
                emb_ref, lg_ref):
    num = acc_ref[0] + acc_ref[1]
    dexp = den_ref[0] + den_ref[1]
    emb = num / (dexp + 1e-16) + b2_ref[...]
    emb_ref[...] = emb
    hc = jnp.maximum(
        jnp.dot(emb, wc1_ref[...], preferred_element_type=jnp.float32, precision=lax.Precision.HIGHEST)
        + bc1_ref[...], 0.0)
    lg_ref[...] = (jnp.dot(hc, wc2_ref[...], preferred_element_type=jnp.float32, precision=lax.Precision.HIGHEST)
                   + bc2_ref[...])


def _final_call(acc, den, b2, wc1, bc1, wc2p, bc2p):
    outs = [
        jax.ShapeDtypeStruct((N, F), jnp.float32),
        jax.ShapeDtypeStruct((N, F), jnp.float32),
    ]
    return pl.pallas_call(
        _final_body,
        grid=(N // _BM,),
        in_specs=[
            pl.BlockSpec((NC, _BM, F), lambda i: (0, i, 0)),
            pl.BlockSpec((NC, _BM, F), lambda i: (0, i, 0)),
            pl.BlockSpec((1, F), lambda i: (0, 0)),
            pl.BlockSpec((F, F), lambda i: (0, 0)),
            pl.BlockSpec((1, F), lambda i: (0, 0)),
            pl.BlockSpec((F, F), lambda i: (0, 0)),
            pl.BlockSpec((1, F), lambda i: (0, 0)),
        ],
        out_specs=[pl.BlockSpec((_BM, F), lambda i: (i, 0))] * 2,
        out_shape=outs,
    )(acc, den, b2, wc1, bc1, wc2p, bc2p)


def _fold_weights(W, att_src, att_dst, heads, ch):
    a_s = att_src.reshape(heads, ch)
    a_d = att_dst.reshape(heads, ch)
    eyeh = jnp.eye(heads, dtype=jnp.float32)
    p_s = (a_s[:, :, None, None] * eyeh[:, None, :, None]
           * jnp.ones((1, 1, 1, ch))).reshape(heads * ch, heads * ch)
    p_d = (a_d[:, :, None, None] * eyeh[:, None, :, None]
           * jnp.ones((1, 1, 1, ch))).reshape(heads * ch, heads * ch)
    mm = lambda a, b: jnp.dot(a, b, precision=lax.Precision.HIGHEST)
    return jnp.concatenate([W, mm(W, p_s), mm(W, p_d)], axis=1)


_num_call = None
_den_call = None


def _get_calls():
    global _num_call, _den_call
    if _num_call is None:
        _num_call = _make_edge_call(True)
        _den_call = _make_edge_call(False)
    return _num_call, _den_call


def kernel(x, edge_index, W1, att_src1, att_dst1, b1, W2, att_src2, att_dst2,
           b2, Wc1, bc1, Wc2, bc2):
    num_call, den_call = _get_calls()
    src = edge_index[0].astype(jnp.int32)
    dst = edge_index[1].astype(jnp.int32)

    wcat1 = _fold_weights(W1, att_src1, att_dst1, HEADS, C1)
    wcat2 = _fold_weights(W2, att_src2, att_dst2, 1, F)

    ah1, asx1, adx1 = _proj_call(x, wcat1)
    acc1 = num_call(src, dst, ah1, adx1)
    den1 = den_call(src, dst, asx1, adx1)

    ah2, asx2, adx2 = _combine_proj_call(acc1, den1, b1.reshape(1, F), wcat2)
    acc2 = num_call(src, dst, ah2, adx2)
    den2 = den_call(src, dst, asx2, adx2)

    wc2p = jnp.zeros((F, F), jnp.float32).at[:, :2].set(Wc2)
    bc2p = jnp.zeros((1, F), jnp.float32).at[0, :2].set(bc2)
    emb, logitsp = _final_call(acc2, den2, b2.reshape(1, F),
                               Wc1, bc1.reshape(1, F), wc2p, bc2p)
    return emb, logitsp[:, :2]


# speedup vs baseline: 1.0404x; 1.0072x over previous
import functools

import jax
import jax.numpy as jnp
from jax import lax
from jax.experimental import pallas as pl
from jax.experimental.pallas import tpu as pltpu
from jax.experimental.pallas import tpu_sc as plsc

N = 10000
E = 320000
F = 128
HEADS = 8
C1 = 16
NC, NS, L = 2, 16, 16
NW = NC * NS
EPW = E // NW
K = 40
NWIN = EPW // K
RPT = 640
ZR = 16


def _edge_kernel_body(with_h, src_hbm, dst_hbm, a_hbm, b_hbm,
                      out_hbm, acc_sh, idxs, idxd, A, B, M, zb,
                      semg, sems, semi):
    c = lax.axis_index("c")
    s = lax.axis_index("s")
    wid = s * NC + c

    def zv(i, _):
        zb[i // (F // L), pl.ds((i % (F // L)) * L, L)] = jnp.zeros(
            (L,), jnp.float32)
        return ()
    lax.fori_loop(0, ZR * (F // L), zv, ())
    for r in range(RPT // ZR):
        off = s * RPT + r * ZR
        @pl.when(off <= N - ZR)
        def _():
            pltpu.sync_copy(zb, acc_sh.at[pl.ds(off, ZR)])
    plsc.subcore_barrier()

    base = wid * EPW

    def issue_idx(w):
        t = w % 3
        off = base + w * K
        pltpu.async_copy(src_hbm.at[pl.ds(off, K)], idxs.at[t], semi.at[t])
        pltpu.async_copy(dst_hbm.at[pl.ds(off, K)], idxd.at[t], semi.at[t])

    def drain_idx(w):
        t = w % 3
        off = base + w * K
        pltpu.make_async_copy(
            src_hbm.at[pl.ds(off, K)], idxs.at[t], semi.at[t]).wait()
        pltpu.make_async_copy(
            dst_hbm.at[pl.ds(off, K)], idxd.at[t], semi.at[t]).wait()

    def issue_gathers(w):
        p, t = w % 2, w % 3
        pltpu.async_copy(a_hbm.at[idxs.at[t]], A.at[p], semg.at[p])
        pltpu.async_copy(b_hbm.at[idxd.at[t]], B.at[p], semg.at[p])

    def drain_gathers(w):
        p, t = w % 2, w % 3
        pltpu.make_async_copy(a_hbm.at[idxs.at[t]], A.at[p], semg.at[p]).wait()
        pltpu.make_async_copy(b_hbm.at[idxd.at[t]], B.at[p], semg.at[p]).wait()

    def drain_scatter(w):
        p, t = w % 2, w % 3
        pltpu.make_async_copy(
            M.at[p], acc_sh.at[idxd.at[t]], sems.at[p]).wait()

    issue_idx(0)
    issue_idx(1)
    drain_idx(0)
    issue_gathers(0)

    def window(w, _):
        p, t = w % 2, w % 3

        @pl.when(w + 2 < NWIN)
        def _():
            issue_idx(w + 2)

        @pl.when(w + 1 < NWIN)
        def _():
            drain_idx(w + 1)
            issue_gathers(w + 1)
        drain_gathers(w)

        @pl.when(w >= 2)
        def _():
            drain_scatter(w - 2)

        @plsc.parallel_loop(0, K * (F // L), unroll=16)
        def edge(i):
            e = i // (F // L)
            j = i % (F // L)
            q = pl.ds(j * L, L)
            z = A[p, e, q] + B[p, e, q]
            z = jnp.maximum(z, 0.2 * z)
            ex = jnp.exp(z)
            if with_h:
                M[p, e, q] = ex * A[p, e, pl.ds(F + j * L, L)]
            else:
                M[p, e, q] = ex
        pltpu.async_copy(M.at[p], acc_sh.at[idxd.at[t]], sems.at[p], add=True)
        return ()
    lax.fori_loop(0, NWIN, window, ())

    for w in (NWIN - 2, NWIN - 1):
        drain_scatter(w)
    plsc.subcore_barrier()

    @pl.when(s < NS - 1)
    def _():
        rows = pl.ds(s * RPT, RPT)
        pltpu.sync_copy(acc_sh.at[rows], out_hbm.at[c, rows])

    @pl.when(s == NS - 1)
    def _():
        rows = pl.ds((NS - 1) * RPT, N - (NS - 1) * RPT)
        pltpu.sync_copy(acc_sh.at[rows], out_hbm.at[c, rows])


def _make_edge_call(with_h):
    CA = 2 * F if with_h else F
    mesh = plsc.VectorSubcoreMesh(core_axis_name="c", subcore_axis_name="s")
    scratch = [
        pltpu.VMEM_SHARED((N, F), jnp.float32),
        pltpu.VMEM((3, K), jnp.int32),
        pltpu.VMEM((3, K), jnp.int32),
        pltpu.VMEM((2, K, CA), jnp.float32),
        pltpu.VMEM((2, K, F), jnp.float32),
        pltpu.VMEM((2, K, F), jnp.float32),
        pltpu.VMEM((ZR, F), jnp.float32),
        pltpu.SemaphoreType.DMA((2,)),
        pltpu.SemaphoreType.DMA((2,)),
        pltpu.SemaphoreType.DMA((3,)),
    ]
    return pl.kernel(
        functools.partial(_edge_kernel_body, with_h),
        out_type=jax.ShapeDtypeStruct((NC, N, F), jnp.float32),
        mesh=mesh,
        scratch_types=scratch,
    )


_BM = 1000


def _proj_body(x_ref, w_ref, ah_ref, ax_ref, bx_ref):
    r = jnp.dot(x_ref[...], w_ref[...], preferred_element_type=jnp.float32, precision=lax.Precision.HIGHEST)
    ah_ref[:, :F] = r[:, F:2 * F]
    ah_ref[:, F:] = r[:, :F]
    ax_ref[...] = r[:, F:2 * F]
    bx_ref[...] = r[:, 2 * F:]


def _proj_call(x, wcat):
    g = x.shape[0] // _BM
    outs = [
        jax.ShapeDtypeStruct((x.shape[0], 2 * F), jnp.float32),
        jax.ShapeDtypeStruct((x.shape[0], F), jnp.float32),
        jax.ShapeDtypeStruct((x.shape[0], F), jnp.float32),
    ]
    return pl.pallas_call(
        _proj_body,
        grid=(g,),
        in_specs=[
            pl.BlockSpec((_BM, F), lambda i: (i, 0)),
            pl.BlockSpec((F, 3 * F), lambda i: (0, 0)),
        ],
        out_specs=[
            pl.BlockSpec((_BM, 2 * F), lambda i: (i, 0)),
            pl.BlockSpec((_BM, F), lambda i: (i, 0)),
            pl.BlockSpec((_BM, F), lambda i: (i, 0)),
        ],
        out_shape=outs,
    )(x, wcat)


def _combine_proj_body(acc_ref, den_ref, b_ref, w_ref, ah_ref, ax_ref, bx_ref):
    num = acc_ref[0] + acc_ref[1]
    dexp = den_ref[0] + den_ref[1]
    g = jnp.maximum(num / (dexp + 1e-16) + b_ref[...], 0.0)
    r = jnp.dot(g, w_ref[...], preferred_element_type=jnp.float32, precision=lax.Precision.HIGHEST)
    ah_ref[:, :F] = r[:, F:2 * F]
    ah_ref[:, F:] = r[:, :F]
    ax_ref[...] = r[:, F:2 * F]
    bx_ref[...] = r[:, 2 * F:]


def _combine_proj_call(acc, den, b1, wcat):
    outs = [
        jax.ShapeDtypeStruct((N, 2 * F), jnp.float32),
        jax.ShapeDtypeStruct((N, F), jnp.float32),
        jax.ShapeDtypeStruct((N, F), jnp.float32),
    ]
    return pl.pallas_call(
        _combine_proj_body,
        grid=(N // _BM,),
        in_specs=[
            pl.BlockSpec((NC, _BM, F), lambda i: (0, i, 0)),
            pl.BlockSpec((NC, _BM, F), lambda i: (0, i, 0)),
            pl.BlockSpec((1, F), lambda i: (0, 0)),
            pl.BlockSpec((F, 3 * F), lambda i: (0, 0)),
        ],
        out_specs=[
            pl.BlockSpec((_BM, 2 * F), lambda i: (i, 0)),
            pl.BlockSpec((_BM, F), lambda i: (i, 0)),
            pl.BlockSpec((_BM, F), lambda i: (i, 0)),
        ],
        out_shape=outs,
    )(acc, den, b1, wcat)


def _final_body(acc_ref, den_ref, b2_ref, wc1_ref, bc1_ref, wc2_ref, bc2_ref,
                emb_ref, lg_ref):
    num = acc_ref[0] + acc_ref[1]
    dexp = den_ref[0] + den_ref[1]
    emb = num / (dexp + 1e-16) + b2_ref[...]
    emb_ref[...] = emb
    hc = jnp.maximum(
        jnp.dot(emb, wc1_ref[...], preferred_element_type=jnp.float32, precision=lax.Precision.HIGHEST)
        + bc1_ref[...], 0.0)
    lg_ref[...] = (jnp.dot(hc, wc2_ref[...], preferred_element_type=jnp.float32, precision=lax.Precision.HIGHEST)
                   + bc2_ref[...])


def _final_call(acc, den, b2, wc1, bc1, wc2p, bc2p):
    outs = [
        jax.ShapeDtypeStruct((N, F), jnp.float32),
        jax.ShapeDtypeStruct((N, F), jnp.float32),
    ]
    return pl.pallas_call(
        _final_body,
        grid=(N // _BM,),
        in_specs=[
            pl.BlockSpec((NC, _BM, F), lambda i: (0, i, 0)),
            pl.BlockSpec((NC, _BM, F), lambda i: (0, i, 0)),
            pl.BlockSpec((1, F), lambda i: (0, 0)),
            pl.BlockSpec((F, F), lambda i: (0, 0)),
            pl.BlockSpec((1, F), lambda i: (0, 0)),
            pl.BlockSpec((F, F), lambda i: (0, 0)),
            pl.BlockSpec((1, F), lambda i: (0, 0)),
        ],
        out_specs=[pl.BlockSpec((_BM, F), lambda i: (i, 0))] * 2,
        out_shape=outs,
    )(acc, den, b2, wc1, bc1, wc2p, bc2p)


def _fold_weights(W, att_src, att_dst, heads, ch):
    a_s = att_src.reshape(heads, ch)
    a_d = att_dst.reshape(heads, ch)
    eyeh = jnp.eye(heads, dtype=jnp.float32)
    p_s = (a_s[:, :, None, None] * eyeh[:, None, :, None]
           * jnp.ones((1, 1, 1, ch))).reshape(heads * ch, heads * ch)
    p_d = (a_d[:, :, None, None] * eyeh[:, None, :, None]
           * jnp.ones((1, 1, 1, ch))).reshape(heads * ch, heads * ch)
    mm = lambda a, b: jnp.dot(a, b, precision=lax.Precision.HIGHEST)
    return jnp.concatenate([W, mm(W, p_s), mm(W, p_d)], axis=1)


_num_call = None
_den_call = None


def _get_calls():
    global _num_call, _den_call
    if _num_call is None:
        _num_call = _make_edge_call(True)
        _den_call = _make_edge_call(False)
    return _num_call, _den_call


def kernel(x, edge_index, W1, att_src1, att_dst1, b1, W2, att_src2, att_dst2,
           b2, Wc1, bc1, Wc2, bc2):
    num_call, den_call = _get_calls()
    src = edge_index[0].astype(jnp.int32)
    dst = edge_index[1].astype(jnp.int32)

    wcat1 = _fold_weights(W1, att_src1, att_dst1, HEADS, C1)
    wcat2 = _fold_weights(W2, att_src2, att_dst2, 1, F)

    ah1, asx1, adx1 = _proj_call(x, wcat1)
    acc1 = num_call(src, dst, ah1, adx1)
    den1 = den_call(src, dst, asx1, adx1)

    ah2, asx2, adx2 = _combine_proj_call(acc1, den1, b1.reshape(1, F), wcat2)
    acc2 = num_call(src, dst, ah2, adx2)
    den2 = den_call(src, dst, asx2, adx2)

    wc2p = jnp.zeros((F, F), jnp.float32).at[:, :2].set(Wc2)
    bc2p = jnp.zeros((1, F), jnp.float32).at[0, :2].set(bc2)
    emb, logitsp = _final_call(acc2, den2, b2.reshape(1, F),
                               Wc1, bc1.reshape(1, F), wc2p, bc2p)
    return emb, logitsp[:, :2]


# speedup vs baseline: 1.0761x; 1.0343x over previous
import functools

import jax
import jax.numpy as jnp
from jax import lax
from jax.experimental import pallas as pl
from jax.experimental.pallas import tpu as pltpu
from jax.experimental.pallas import tpu_sc as plsc

N = 10000
E = 320000
F = 128
HEADS = 8
C1 = 16
NC, NS, L = 2, 16, 16
NW = NC * NS
EPW = E // NW
K = 40
NWIN = EPW // K
RPT = 640
ZR = 16


def _edge_kernel_body(with_h, src_hbm, dst_hbm, a_hbm, b_hbm,
                      out_hbm, acc_sh, idxs, idxd, A, B, M, zb,
                      semg, sems, semi):
    c = lax.axis_index("c")
    s = lax.axis_index("s")
    wid = s * NC + c

    def zv(i, _):
        zb[i // (F // L), pl.ds((i % (F // L)) * L, L)] = jnp.zeros(
            (L,), jnp.float32)
        return ()
    lax.fori_loop(0, ZR * (F // L), zv, ())
    for r in range(RPT // ZR):
        off = s * RPT + r * ZR
        @pl.when(off <= N - ZR)
        def _():
            pltpu.sync_copy(zb, acc_sh.at[pl.ds(off, ZR)])
    plsc.subcore_barrier()

    base = wid * EPW

    def issue_idx(w):
        t = w % 3
        off = base + w * K
        pltpu.async_copy(src_hbm.at[pl.ds(off, K)], idxs.at[t], semi.at[t])
        pltpu.async_copy(dst_hbm.at[pl.ds(off, K)], idxd.at[t], semi.at[t])

    def drain_idx(w):
        t = w % 3
        off = base + w * K
        pltpu.make_async_copy(
            src_hbm.at[pl.ds(off, K)], idxs.at[t], semi.at[t]).wait()
        pltpu.make_async_copy(
            dst_hbm.at[pl.ds(off, K)], idxd.at[t], semi.at[t]).wait()

    def issue_gathers(w):
        p, t = w % 2, w % 3
        pltpu.async_copy(a_hbm.at[idxs.at[t]], A.at[p], semg.at[p])
        pltpu.async_copy(b_hbm.at[idxd.at[t]], B.at[p], semg.at[p])

    def drain_gathers(w):
        p, t = w % 2, w % 3
        pltpu.make_async_copy(a_hbm.at[idxs.at[t]], A.at[p], semg.at[p]).wait()
        pltpu.make_async_copy(b_hbm.at[idxd.at[t]], B.at[p], semg.at[p]).wait()

    def drain_scatter(w):
        p, t = w % 2, w % 3
        pltpu.make_async_copy(
            M.at[p], acc_sh.at[idxd.at[t]], sems.at[p]).wait()

    issue_idx(0)
    issue_idx(1)
    drain_idx(0)
    issue_gathers(0)

    def window(w, _):
        p, t = w % 2, w % 3

        @pl.when(w + 2 < NWIN)
        def _():
            issue_idx(w + 2)

        @pl.when(w + 1 < NWIN)
        def _():
            drain_idx(w + 1)
            issue_gathers(w + 1)
        drain_gathers(w)

        @pl.when(w >= 2)
        def _():
            drain_scatter(w - 2)

        @plsc.parallel_loop(0, K * (F // L), unroll=16)
        def edge(i):
            e = i // (F // L)
            j = i % (F // L)
            q = pl.ds(j * L, L)
            z = A[p, e, q] + B[p, e, q]
            z = jnp.maximum(z, 0.2 * z)
            ex = jnp.exp(z)
            if with_h:
                M[p, e, q] = ex * A[p, e, pl.ds(F + j * L, L)]
            else:
                M[p, e, q] = ex
        pltpu.async_copy(M.at[p], acc_sh.at[idxd.at[t]], sems.at[p], add=True)
        return ()
    lax.fori_loop(0, NWIN, window, ())

    for w in (NWIN - 2, NWIN - 1):
        drain_scatter(w)
    plsc.subcore_barrier()

    @pl.when(s < NS - 1)
    def _():
        rows = pl.ds(s * RPT, RPT)
        pltpu.sync_copy(acc_sh.at[rows], out_hbm.at[c, rows])

    @pl.when(s == NS - 1)
    def _():
        rows = pl.ds((NS - 1) * RPT, N - (NS - 1) * RPT)
        pltpu.sync_copy(acc_sh.at[rows], out_hbm.at[c, rows])


def _make_edge_call(with_h):
    CA = 2 * F if with_h else F
    mesh = plsc.VectorSubcoreMesh(core_axis_name="c", subcore_axis_name="s")
    scratch = [
        pltpu.VMEM_SHARED((N, F), jnp.float32),
        pltpu.VMEM((3, K), jnp.int32),
        pltpu.VMEM((3, K), jnp.int32),
        pltpu.VMEM((2, K, CA), jnp.float32),
        pltpu.VMEM((2, K, F), jnp.float32),
        pltpu.VMEM((2, K, F), jnp.float32),
        pltpu.VMEM((ZR, F), jnp.float32),
        pltpu.SemaphoreType.DMA((2,)),
        pltpu.SemaphoreType.DMA((2,)),
        pltpu.SemaphoreType.DMA((3,)),
    ]
    return pl.kernel(
        functools.partial(_edge_kernel_body, with_h),
        out_type=jax.ShapeDtypeStruct((NC, N, F), jnp.float32),
        mesh=mesh,
        scratch_types=scratch,
    )


_BM = 1000


def _proj_body(x_ref, w_ref, ah_ref, ax_ref, bx_ref):
    r = jnp.dot(x_ref[...], w_ref[...], preferred_element_type=jnp.float32)
    ah_ref[:, :F] = r[:, F:2 * F]
    ah_ref[:, F:] = r[:, :F]
    ax_ref[...] = r[:, F:2 * F]
    bx_ref[...] = r[:, 2 * F:]


def _proj_call(x, wcat):
    g = x.shape[0] // _BM
    outs = [
        jax.ShapeDtypeStruct((x.shape[0], 2 * F), jnp.float32),
        jax.ShapeDtypeStruct((x.shape[0], F), jnp.float32),
        jax.ShapeDtypeStruct((x.shape[0], F), jnp.float32),
    ]
    return pl.pallas_call(
        _proj_body,
        grid=(g,),
        in_specs=[
            pl.BlockSpec((_BM, F), lambda i: (i, 0)),
            pl.BlockSpec((F, 3 * F), lambda i: (0, 0)),
        ],
        out_specs=[
            pl.BlockSpec((_BM, 2 * F), lambda i: (i, 0)),
            pl.BlockSpec((_BM, F), lambda i: (i, 0)),
            pl.BlockSpec((_BM, F), lambda i: (i, 0)),
        ],
        out_shape=outs,
    )(x, wcat)


def _combine_proj_body(acc_ref, den_ref, b_ref, w_ref, ah_ref, ax_ref, bx_ref):
    num = acc_ref[0] + acc_ref[1]
    dexp = den_ref[0] + den_ref[1]
    g = jnp.maximum(num / (dexp + 1e-16) + b_ref[...], 0.0)
    r = jnp.dot(g, w_ref[...], preferred_element_type=jnp.float32)
    ah_ref[:, :F] = r[:, F:2 * F]
    ah_ref[:, F:] = r[:, :F]
    ax_ref[...] = r[:, F:2 * F]
    bx_ref[...] = r[:, 2 * F:]


def _combine_proj_call(acc, den, b1, wcat):
    outs = [
        jax.ShapeDtypeStruct((N, 2 * F), jnp.float32),
        jax.ShapeDtypeStruct((N, F), jnp.float32),
        jax.ShapeDtypeStruct((N, F), jnp.float32),
    ]
    return pl.pallas_call(
        _combine_proj_body,
        grid=(N // _BM,),
        in_specs=[
            pl.BlockSpec((NC, _BM, F), lambda i: (0, i, 0)),
            pl.BlockSpec((NC, _BM, F), lambda i: (0, i, 0)),
            pl.BlockSpec((1, F), lambda i: (0, 0)),
            pl.BlockSpec((F, 3 * F), lambda i: (0, 0)),
        ],
        out_specs=[
            pl.BlockSpec((_BM, 2 * F), lambda i: (i, 0)),
            pl.BlockSpec((_BM, F), lambda i: (i, 0)),
            pl.BlockSpec((_BM, F), lambda i: (i, 0)),
        ],
        out_shape=outs,
    )(acc, den, b1, wcat)


def _final_body(acc_ref, den_ref, b2_ref, wc1_ref, bc1_ref, wc2_ref, bc2_ref,
                emb_ref, lg_ref):
    num = acc_ref[0] + acc_ref[1]
    dexp = den_ref[0] + den_ref[1]
    emb = num / (dexp + 1e-16) + b2_ref[...]
    emb_ref[...] = emb
    hc = jnp.maximum(
        jnp.dot(emb, wc1_ref[...], preferred_element_type=jnp.float32)
        + bc1_ref[...], 0.0)
    lg_ref[...] = (jnp.dot(hc, wc2_ref[...], preferred_element_type=jnp.float32)
                   + bc2_ref[...])


def _final_call(acc, den, b2, wc1, bc1, wc2p, bc2p):
    outs = [
        jax.ShapeDtypeStruct((N, F), jnp.float32),
        jax.ShapeDtypeStruct((N, F), jnp.float32),
    ]
    return pl.pallas_call(
        _final_body,
        grid=(N // _BM,),
        in_specs=[
            pl.BlockSpec((NC, _BM, F), lambda i: (0, i, 0)),
            pl.BlockSpec((NC, _BM, F), lambda i: (0, i, 0)),
            pl.BlockSpec((1, F), lambda i: (0, 0)),
            pl.BlockSpec((F, F), lambda i: (0, 0)),
            pl.BlockSpec((1, F), lambda i: (0, 0)),
            pl.BlockSpec((F, F), lambda i: (0, 0)),
            pl.BlockSpec((1, F), lambda i: (0, 0)),
        ],
        out_specs=[pl.BlockSpec((_BM, F), lambda i: (i, 0))] * 2,
        out_shape=outs,
    )(acc, den, b2, wc1, bc1, wc2p, bc2p)


def _fold_weights(W, att_src, att_dst, heads, ch):
    a_s = att_src.reshape(heads, ch)
    a_d = att_dst.reshape(heads, ch)
    eyeh = jnp.eye(heads, dtype=jnp.float32)
    p_s = (a_s[:, :, None, None] * eyeh[:, None, :, None]
           * jnp.ones((1, 1, 1, ch))).reshape(heads * ch, heads * ch)
    p_d = (a_d[:, :, None, None] * eyeh[:, None, :, None]
           * jnp.ones((1, 1, 1, ch))).reshape(heads * ch, heads * ch)
    mm = lambda a, b: jnp.dot(a, b)
    return jnp.concatenate([W, mm(W, p_s), mm(W, p_d)], axis=1)


_num_call = None
_den_call = None


def _get_calls():
    global _num_call, _den_call
    if _num_call is None:
        _num_call = _make_edge_call(True)
        _den_call = _make_edge_call(False)
    return _num_call, _den_call


def kernel(x, edge_index, W1, att_src1, att_dst1, b1, W2, att_src2, att_dst2,
           b2, Wc1, bc1, Wc2, bc2):
    num_call, den_call = _get_calls()
    src = edge_index[0].astype(jnp.int32)
    dst = edge_index[1].astype(jnp.int32)

    wcat1 = _fold_weights(W1, att_src1, att_dst1, HEADS, C1)
    wcat2 = _fold_weights(W2, att_src2, att_dst2, 1, F)

    ah1, asx1, adx1 = _proj_call(x, wcat1)
    acc1 = num_call(src, dst, ah1, adx1)
    den1 = den_call(src, dst, asx1, adx1)

    ah2, asx2, adx2 = _combine_proj_call(acc1, den1, b1.reshape(1, F), wcat2)
    acc2 = num_call(src, dst, ah2, adx2)
    den2 = den_call(src, dst, asx2, adx2)

    wc2p = jnp.zeros((F, F), jnp.float32).at[:, :2].set(Wc2)
    bc2p = jnp.zeros((1, F), jnp.float32).at[0, :2].set(bc2)
    emb, logitsp = _final_call(acc2, den2, b2.reshape(1, F),
                               Wc1, bc1.reshape(1, F), wc2p, bc2p)
    return emb, logitsp[:, :2]
